# baseline (speedup 1.0000x reference)
import functools

import jax
import jax.numpy as jnp
from jax import lax
from jax.experimental import pallas as pl
from jax.experimental.pallas import tpu as pltpu
from jax.experimental.pallas import tpu_sc as plsc

D_MODEL = 1024
D_FF = 2048
N_EXP = 8
T = 2048
K = 2
NS = 4096
BLK = 256
NB = 24
NSORT = NB * BLK
NEG = -1e30

NC = 2
NSUB = 16
NW = NC * NSUB
ROWS_B = NS // NW
ROWS_D = T // NW
CHB = 32
CHD = 16
NSETS = 2


def _cumsum_rows(a):
    n = a.shape[0]
    s = 1
    while s < n:
        sh = jnp.concatenate(
            [jnp.zeros((s, a.shape[1]), a.dtype), a[:n - s]], axis=0)
        a = a + sh
        s *= 2
    return a


def _cumsum_lanes(a):
    n = a.shape[1]
    s = 1
    while s < n:
        sh = jnp.concatenate(
            [jnp.zeros((a.shape[0], s), a.dtype), a[:, :n - s]], axis=1)
        a = a + sh
        s *= 2
    return a


def _gating_kernel(x_ref, wg_ref, bg_ref, pos_ref, prob_ref, beidx_ref,
                   beact_ref, isf_ref, par_ref, fe_ref, hnext_ref):
    g = jnp.dot(x_ref[...], wg_ref[...],
                preferred_element_type=jnp.float32) + bg_ref[0, :]
    idx8 = lax.broadcasted_iota(jnp.int32, (T, N_EXP), 1)
    m0 = jnp.max(g, axis=1, keepdims=True)
    e0 = jnp.min(jnp.where(g >= m0, idx8, N_EXP), axis=1, keepdims=True)
    gm = jnp.where(idx8 == e0, NEG, g)
    m1 = jnp.max(gm, axis=1, keepdims=True)
    e1 = jnp.min(jnp.where(gm >= m1, idx8, N_EXP), axis=1, keepdims=True)
    p0 = 1.0 / (1.0 + jnp.exp(m1 - m0))
    p1 = 1.0 - p0

    oh0 = (idx8 == e0).astype(jnp.float32)
    oh1 = (idx8 == e1).astype(jnp.float32)
    c01 = _cumsum_rows(jnp.concatenate([oh0, oh1], axis=1))
    c0 = c01[:, :N_EXP]
    c1 = c01[:, N_EXP:]
    cnt0 = c0[T - 1:T, :]
    cnt = cnt0 + c1[T - 1:T, :]
    padded = jnp.ceil(cnt * (1.0 / BLK)) * BLK
    off = _cumsum_lanes(padded) - padded

    r0 = jnp.sum(oh0 * (off + c0), axis=1, keepdims=True) - 1.0
    r1 = jnp.sum(oh1 * (off + cnt0 + c1), axis=1, keepdims=True) - 1.0
    pos_ref[0:T, :] = r0.astype(jnp.int32)
    pos_ref[T:NS, :] = r1.astype(jnp.int32)

    prob_ref[0:T, :] = jnp.broadcast_to(p0, (T, 128))
    prob_ref[T:NS, :] = jnp.broadcast_to(p1, (T, 128))

    ends = off + padded
    total = jnp.sum(padded, axis=1, keepdims=True)
    nbv = lax.broadcasted_iota(jnp.int32, (NB, 1), 0).astype(jnp.float32) * BLK
    be = jnp.sum((nbv >= ends).astype(jnp.float32), axis=1, keepdims=True)
    act = (nbv < total)
    bi = jnp.minimum(be, N_EXP - 1)
    beidx_ref[...] = bi.astype(jnp.int32)
    beact_ref[...] = act.astype(jnp.int32)

    iota_e = lax.broadcasted_iota(jnp.int32, (NB, N_EXP), 1).astype(jnp.float32)
    oh_b = (iota_e == bi).astype(jnp.float32)
    off_b = jnp.sum(oh_b * off, axis=1, keepdims=True)
    nonempty = (padded > 0).astype(jnp.float32)
    rid_of_f = _cumsum_lanes(nonempty) - nonempty
    rid_b = jnp.sum(oh_b * rid_of_f, axis=1, keepdims=True)
    nruns = jnp.sum(nonempty, axis=1, keepdims=True)
    eidx = lax.broadcasted_iota(jnp.int32, (1, N_EXP), 1).astype(jnp.float32)
    fe = jnp.sum((rid_of_f == rid_b + 1.0) * nonempty * eidx, axis=1,
                 keepdims=True)
    isf_ref[...] = ((nbv == off_b).astype(jnp.float32)
                    * act.astype(jnp.float32)).astype(jnp.int32)
    par_ref[...] = (rid_b - 2.0 * jnp.floor(rid_b * 0.5)).astype(jnp.int32)
    fe_ref[...] = fe.astype(jnp.int32)
    hnext_ref[...] = (rid_b + 1.0 < nruns).astype(jnp.int32)


def _gating(x2d, Wg, bg2):
    return pl.pallas_call(
        _gating_kernel,
        grid=(1,),
        in_specs=[
            pl.BlockSpec((T, D_MODEL), lambda i: (0, 0)),
            pl.BlockSpec((D_MODEL, N_EXP), lambda i: (0, 0)),
            pl.BlockSpec((1, N_EXP), lambda i: (0, 0)),
        ],
        out_specs=[
            pl.BlockSpec((NS, 1), lambda i: (0, 0)),
            pl.BlockSpec((NS, 128), lambda i: (0, 0)),
            pl.BlockSpec((NB, 1), lambda i: (0, 0)),
            pl.BlockSpec((NB, 1), lambda i: (0, 0)),
            pl.BlockSpec((NB, 1), lambda i: (0, 0)),
            pl.BlockSpec((NB, 1), lambda i: (0, 0)),
            pl.BlockSpec((NB, 1), lambda i: (0, 0)),
            pl.BlockSpec((NB, 1), lambda i: (0, 0)),
        ],
        out_shape=[
            jax.ShapeDtypeStruct((NS, 1), jnp.int32),
            jax.ShapeDtypeStruct((NS, 128), jnp.float32),
        ] + [jax.ShapeDtypeStruct((NB, 1), jnp.int32)] * 6,
    )(x2d, Wg, bg2)


@functools.lru_cache(maxsize=None)
def _make_scatter():
    mesh = plsc.VectorSubcoreMesh(core_axis_name="c", subcore_axis_name="s")

    nch = ROWS_B // CHB

    @functools.partial(
        pl.kernel,
        mesh=mesh,
        out_type=[
            jax.ShapeDtypeStruct((NSORT, D_MODEL), jnp.float32),
            jax.ShapeDtypeStruct((NSORT, 128), jnp.float32),
        ],
        scratch_types=(
            [pltpu.VMEM((CHB,), jnp.int32)] * NSETS
            + [pltpu.VMEM((CHB, D_MODEL), jnp.float32)] * NSETS
            + [pltpu.VMEM((CHB, 128), jnp.float32)] * NSETS
            + [pltpu.SemaphoreType.DMA] * NSETS
            + [pltpu.SemaphoreType.DMA] * NSETS
        ),
    )
    def scatter_k(x_hbm, pos_hbm, prob_hbm, xs_hbm, ps_hbm, *scr):
        idx_v = scr[0:NSETS]
        row_v = scr[NSETS:2 * NSETS]
        prb_v = scr[2 * NSETS:3 * NSETS]
        lsem = scr[3 * NSETS:4 * NSETS]
        ssem = scr[4 * NSETS:5 * NSETS]
        wid = lax.axis_index("s") * NC + lax.axis_index("c")
        slot_base = wid * ROWS_B
        tok_base = (wid % NSUB) * ROWS_B

        def load(c):
            s = c % NSETS
            sb = slot_base + c * CHB
            tb = tok_base + c * CHB
            return (
                pltpu.async_copy(pos_hbm.at[pl.ds(sb, CHB)], idx_v[s],
                                 lsem[s]),
                pltpu.async_copy(x_hbm.at[pl.ds(tb, CHB)], row_v[s], lsem[s]),
                pltpu.async_copy(prob_hbm.at[pl.ds(sb, CHB)], prb_v[s],
                                 lsem[s]),
            )

        def store(c):
            s = c % NSETS
            return (
                pltpu.async_copy(row_v[s], xs_hbm.at[idx_v[s]], ssem[s]),
                pltpu.async_copy(prb_v[s], ps_hbm.at[idx_v[s]], ssem[s]),
            )

        loads = {c: load(c) for c in range(NSETS)}
        stores = {}
        for c in range(nch):
            for cp in loads[c]:
                cp.wait()
            stores[c] = store(c)
            if c + NSETS < nch:
                for cp in stores[c]:
                    cp.wait()
                stores.pop(c)
                loads[c + NSETS] = load(c + NSETS)
        for c in stores:
            for cp in stores[c]:
                cp.wait()

    return scatter_k


W1CH = D_MODEL // 8
W2CH = D_FF // 8


def _ffn_kernel(beidx_ref, beact_ref, isf_ref, par_ref, fe_ref, hnext_ref,
                x_ref, w1_ref, b1_ref, w2_ref, b2_ref, prob_ref, y_ref,
                w1b_ref, w2b_ref, wsem):
    nb = pl.program_id(0)
    act = beact_ref[nb] > 0

    def issue(e, p):
        for ch in range(8):
            pltpu.make_async_copy(
                w1_ref.at[e, pl.ds(ch * W1CH, W1CH), :],
                w1b_ref.at[p, pl.ds(ch * W1CH, W1CH), :], wsem).start()
            pltpu.make_async_copy(
                w2_ref.at[e, pl.ds(ch * W2CH, W2CH), :],
                w2b_ref.at[p, pl.ds(ch * W2CH, W2CH), :], wsem).start()

    def drain(e, p):
        for ch in range(8):
            pltpu.make_async_copy(
                w1_ref.at[e, pl.ds(ch * W1CH, W1CH), :],
                w1b_ref.at[p, pl.ds(ch * W1CH, W1CH), :], wsem).wait()
            pltpu.make_async_copy(
                w2_ref.at[e, pl.ds(ch * W2CH, W2CH), :],
                w2b_ref.at[p, pl.ds(ch * W2CH, W2CH), :], wsem).wait()

    @pl.when(jnp.logical_and(act, nb == 0))
    def _():
        issue(beidx_ref[0], par_ref[0])

    @pl.when(jnp.logical_and(act, isf_ref[nb] > 0))
    def _():
        e = beidx_ref[nb]
        p = par_ref[nb]
        drain(e, p)

        @pl.when(hnext_ref[nb] > 0)
        def _():
            issue(fe_ref[nb], 1 - p)

    @pl.when(act)
    def _():
        p = par_ref[nb]
        h = jnp.dot(x_ref[...], w1b_ref[p], preferred_element_type=jnp.float32)
        h = h + b1_ref[0, 0, :]
        h = 0.5 * h * (1.0 + lax.erf(h * 0.7071067811865476))
        y = jnp.dot(h, w2b_ref[p],
                    preferred_element_type=jnp.float32) + b2_ref[0, 0, :]
        y_ref[...] = y * prob_ref[:, 0:1]


def _ffn(beidx, beact, isf, par, fe, hnext, xs, w1f, b1r, w2f, b2r, ps):
    grid_spec = pltpu.PrefetchScalarGridSpec(
        num_scalar_prefetch=6,
        grid=(NB,),
        in_specs=[
            pl.BlockSpec((BLK, D_MODEL), lambda nb, *_: (nb, 0)),
            pl.BlockSpec(memory_space=pl.ANY),
            pl.BlockSpec((1, 1, D_FF), lambda nb, bi, *_: (bi[nb], 0, 0)),
            pl.BlockSpec(memory_space=pl.ANY),
            pl.BlockSpec((1, 1, D_MODEL), lambda nb, bi, *_: (bi[nb], 0, 0)),
            pl.BlockSpec((BLK, 128), lambda nb, *_: (nb, 0)),
        ],
        out_specs=pl.BlockSpec((BLK, D_MODEL), lambda nb, *_: (nb, 0)),
        scratch_shapes=[
            pltpu.VMEM((2, D_MODEL, D_FF), jnp.float32),
            pltpu.VMEM((2, D_FF, D_MODEL), jnp.float32),
            pltpu.SemaphoreType.DMA,
        ],
    )
    return pl.pallas_call(
        _ffn_kernel,
        grid_spec=grid_spec,
        out_shape=jax.ShapeDtypeStruct((NSORT, D_MODEL), jnp.float32),
    )(beidx, beact, isf, par, fe, hnext, xs, w1f, b1r, w2f, b2r, ps)


@functools.lru_cache(maxsize=None)
def _make_combine():
    mesh = plsc.VectorSubcoreMesh(core_axis_name="c", subcore_axis_name="s")

    nch = ROWS_D // CHD

    @functools.partial(
        pl.kernel,
        mesh=mesh,
        out_type=jax.ShapeDtypeStruct((T, D_MODEL), jnp.float32),
        scratch_types=(
            [pltpu.VMEM((CHD,), jnp.int32)] * (2 * NSETS)
            + [pltpu.VMEM((CHD, D_MODEL), jnp.float32)] * (2 * NSETS)
            + [pltpu.SemaphoreType.DMA] * NSETS
            + [pltpu.SemaphoreType.DMA] * NSETS
        ),
    )
    def combine_k(y_hbm, pos_hbm, o_hbm, *scr):
        i0_v = scr[0:NSETS]
        i1_v = scr[NSETS:2 * NSETS]
        g0_v = scr[2 * NSETS:3 * NSETS]
        g1_v = scr[3 * NSETS:4 * NSETS]
        gsem = scr[4 * NSETS:5 * NSETS]
        ssem = scr[5 * NSETS:6 * NSETS]
        wid = lax.axis_index("s") * NC + lax.axis_index("c")
        tok_base = wid * ROWS_D

        def gather(c):
            s = c % NSETS
            tb = tok_base + c * CHD
            pltpu.sync_copy(pos_hbm.at[pl.ds(tb, CHD)], i0_v[s])
            pltpu.sync_copy(pos_hbm.at[pl.ds(T + tb, CHD)], i1_v[s])
            return (
                pltpu.async_copy(y_hbm.at[i0_v[s]], g0_v[s], gsem[s]),
                pltpu.async_copy(y_hbm.at[i1_v[s]], g1_v[s], gsem[s]),
            )

        gathers = {c: gather(c) for c in range(NSETS)}
        stores = {}
        for c in range(nch):
            s = c % NSETS
            tb = tok_base + c * CHD
            for cp in gathers[c]:
                cp.wait()

            @pl.loop(0, CHD)
            def _(r):
                @pl.loop(0, D_MODEL, step=16)
                def _(cc):
                    sl = (pl.ds(r, 1), pl.ds(cc, 16))
                    g0_v[s].at[*sl][...] = (g0_v[s].at[*sl][...]
                                            + g1_v[s].at[*sl][...])

            stores[c] = pltpu.async_copy(g0_v[s], o_hbm.at[pl.ds(tb, CHD)],
                                         ssem[s])
            if c + NSETS < nch:
                stores[c].wait()
                stores.pop(c)
                gathers[c + NSETS] = gather(c + NSETS)
        for c in stores:
            stores[c].wait()

    return combine_k


def kernel(x, W1, b1, W2, b2, Wg, bg):
    B, S, D = x.shape
    x2d = x.reshape(T, D)
    b1r = b1.reshape(N_EXP, 1, D_FF)
    b2r = b2.reshape(N_EXP, 1, D_MODEL)
    bg2 = bg.reshape(1, N_EXP)

    (pos, prob, beidx, beact, isf, par, fe, hnext) = _gating(x2d, Wg, bg2)
    pos1d = pos.reshape(NS)
    xs, ps = _make_scatter()(x2d, pos1d, prob)
    ys = _ffn(beidx.reshape(NB), beact.reshape(NB), isf.reshape(NB),
              par.reshape(NB), fe.reshape(NB), hnext.reshape(NB), xs, W1,
              b1r, W2, b2r, ps)
    out = _make_combine()(ys, pos1d)
    return out.reshape(B, S, D)


# speedup vs baseline: 1.0369x; 1.0369x over previous
import functools

import jax
import jax.numpy as jnp
from jax import lax
from jax.experimental import pallas as pl
from jax.experimental.pallas import tpu as pltpu
from jax.experimental.pallas import tpu_sc as plsc

D_MODEL = 1024
D_FF = 2048
N_EXP = 8
T = 2048
K = 2
NS = 4096
BLK = 384
NB = 18
NSORT = NB * BLK
NEG = -1e30

NC = 2
NSUB = 16
NW = NC * NSUB
ROWS_B = NS // NW
ROWS_D = T // NW
CHB = 32
CHD = 16
NSETS = 2


def _cumsum_rows(a):
    n = a.shape[0]
    s = 1
    while s < n:
        sh = jnp.concatenate(
            [jnp.zeros((s, a.shape[1]), a.dtype), a[:n - s]], axis=0)
        a = a + sh
        s *= 2
    return a


def _cumsum_lanes(a):
    n = a.shape[1]
    s = 1
    while s < n:
        sh = jnp.concatenate(
            [jnp.zeros((a.shape[0], s), a.dtype), a[:, :n - s]], axis=1)
        a = a + sh
        s *= 2
    return a


def _gating_kernel(x_ref, wg_ref, bg_ref, pos_ref, prob_ref, beidx_ref,
                   beact_ref, isf_ref, par_ref, fe_ref, hnext_ref):
    g = jnp.dot(x_ref[...], wg_ref[...],
                preferred_element_type=jnp.float32) + bg_ref[0, :]
    idx8 = lax.broadcasted_iota(jnp.int32, (T, N_EXP), 1)
    m0 = jnp.max(g, axis=1, keepdims=True)
    e0 = jnp.min(jnp.where(g >= m0, idx8, N_EXP), axis=1, keepdims=True)
    gm = jnp.where(idx8 == e0, NEG, g)
    m1 = jnp.max(gm, axis=1, keepdims=True)
    e1 = jnp.min(jnp.where(gm >= m1, idx8, N_EXP), axis=1, keepdims=True)
    p0 = 1.0 / (1.0 + jnp.exp(m1 - m0))
    p1 = 1.0 - p0

    oh0 = (idx8 == e0).astype(jnp.float32)
    oh1 = (idx8 == e1).astype(jnp.float32)
    c01 = _cumsum_rows(jnp.concatenate([oh0, oh1], axis=1))
    c0 = c01[:, :N_EXP]
    c1 = c01[:, N_EXP:]
    cnt0 = c0[T - 1:T, :]
    cnt = cnt0 + c1[T - 1:T, :]
    padded = jnp.ceil(cnt * (1.0 / BLK)) * BLK
    off = _cumsum_lanes(padded) - padded

    r0 = jnp.sum(oh0 * (off + c0), axis=1, keepdims=True) - 1.0
    r1 = jnp.sum(oh1 * (off + cnt0 + c1), axis=1, keepdims=True) - 1.0
    pos_ref[0:T, :] = r0.astype(jnp.int32)
    pos_ref[T:NS, :] = r1.astype(jnp.int32)

    prob_ref[0:T, :] = jnp.broadcast_to(p0, (T, 128))
    prob_ref[T:NS, :] = jnp.broadcast_to(p1, (T, 128))

    ends = off + padded
    total = jnp.sum(padded, axis=1, keepdims=True)
    nbv = lax.broadcasted_iota(jnp.int32, (NB, 1), 0).astype(jnp.float32) * BLK
    be = jnp.sum((nbv >= ends).astype(jnp.float32), axis=1, keepdims=True)
    act = (nbv < total)
    bi = jnp.minimum(be, N_EXP - 1)
    beidx_ref[...] = bi.astype(jnp.int32)
    beact_ref[...] = act.astype(jnp.int32)

    iota_e = lax.broadcasted_iota(jnp.int32, (NB, N_EXP), 1).astype(jnp.float32)
    oh_b = (iota_e == bi).astype(jnp.float32)
    off_b = jnp.sum(oh_b * off, axis=1, keepdims=True)
    nonempty = (padded > 0).astype(jnp.float32)
    rid_of_f = _cumsum_lanes(nonempty) - nonempty
    rid_b = jnp.sum(oh_b * rid_of_f, axis=1, keepdims=True)
    nruns = jnp.sum(nonempty, axis=1, keepdims=True)
    eidx = lax.broadcasted_iota(jnp.int32, (1, N_EXP), 1).astype(jnp.float32)
    fe = jnp.sum((rid_of_f == rid_b + 1.0) * nonempty * eidx, axis=1,
                 keepdims=True)
    isf_ref[...] = ((nbv == off_b).astype(jnp.float32)
                    * act.astype(jnp.float32)).astype(jnp.int32)
    par_ref[...] = (rid_b - 2.0 * jnp.floor(rid_b * 0.5)).astype(jnp.int32)
    fe_ref[...] = fe.astype(jnp.int32)
    hnext_ref[...] = (rid_b + 1.0 < nruns).astype(jnp.int32)


def _gating(x2d, Wg, bg2):
    return pl.pallas_call(
        _gating_kernel,
        grid=(1,),
        in_specs=[
            pl.BlockSpec((T, D_MODEL), lambda i: (0, 0)),
            pl.BlockSpec((D_MODEL, N_EXP), lambda i: (0, 0)),
            pl.BlockSpec((1, N_EXP), lambda i: (0, 0)),
        ],
        out_specs=[
            pl.BlockSpec((NS, 1), lambda i: (0, 0)),
            pl.BlockSpec((NS, 128), lambda i: (0, 0)),
            pl.BlockSpec((NB, 1), lambda i: (0, 0)),
            pl.BlockSpec((NB, 1), lambda i: (0, 0)),
            pl.BlockSpec((NB, 1), lambda i: (0, 0)),
            pl.BlockSpec((NB, 1), lambda i: (0, 0)),
            pl.BlockSpec((NB, 1), lambda i: (0, 0)),
            pl.BlockSpec((NB, 1), lambda i: (0, 0)),
        ],
        out_shape=[
            jax.ShapeDtypeStruct((NS, 1), jnp.int32),
            jax.ShapeDtypeStruct((NS, 128), jnp.float32),
        ] + [jax.ShapeDtypeStruct((NB, 1), jnp.int32)] * 6,
    )(x2d, Wg, bg2)


@functools.lru_cache(maxsize=None)
def _make_scatter():
    mesh = plsc.VectorSubcoreMesh(core_axis_name="c", subcore_axis_name="s")

    nch = ROWS_B // CHB

    @functools.partial(
        pl.kernel,
        mesh=mesh,
        out_type=[
            jax.ShapeDtypeStruct((NSORT, D_MODEL), jnp.float32),
            jax.ShapeDtypeStruct((NSORT, 128), jnp.float32),
        ],
        scratch_types=(
            [pltpu.VMEM((CHB,), jnp.int32)] * NSETS
            + [pltpu.VMEM((CHB, D_MODEL), jnp.float32)] * NSETS
            + [pltpu.VMEM((CHB, 128), jnp.float32)] * NSETS
            + [pltpu.SemaphoreType.DMA] * NSETS
            + [pltpu.SemaphoreType.DMA] * NSETS
        ),
    )
    def scatter_k(x_hbm, pos_hbm, prob_hbm, xs_hbm, ps_hbm, *scr):
        idx_v = scr[0:NSETS]
        row_v = scr[NSETS:2 * NSETS]
        prb_v = scr[2 * NSETS:3 * NSETS]
        lsem = scr[3 * NSETS:4 * NSETS]
        ssem = scr[4 * NSETS:5 * NSETS]
        wid = lax.axis_index("s") * NC + lax.axis_index("c")
        slot_base = wid * ROWS_B
        tok_base = (wid % NSUB) * ROWS_B

        def load(c):
            s = c % NSETS
            sb = slot_base + c * CHB
            tb = tok_base + c * CHB
            return (
                pltpu.async_copy(pos_hbm.at[pl.ds(sb, CHB)], idx_v[s],
                                 lsem[s]),
                pltpu.async_copy(x_hbm.at[pl.ds(tb, CHB)], row_v[s], lsem[s]),
                pltpu.async_copy(prob_hbm.at[pl.ds(sb, CHB)], prb_v[s],
                                 lsem[s]),
            )

        def store(c):
            s = c % NSETS
            return (
                pltpu.async_copy(row_v[s], xs_hbm.at[idx_v[s]], ssem[s]),
                pltpu.async_copy(prb_v[s], ps_hbm.at[idx_v[s]], ssem[s]),
            )

        loads = {c: load(c) for c in range(NSETS)}
        stores = {}
        for c in range(nch):
            for cp in loads[c]:
                cp.wait()
            stores[c] = store(c)
            if c + NSETS < nch:
                for cp in stores[c]:
                    cp.wait()
                stores.pop(c)
                loads[c + NSETS] = load(c + NSETS)
        for c in stores:
            for cp in stores[c]:
                cp.wait()

    return scatter_k


W1CH = D_MODEL // 8
W2CH = D_FF // 8


def _ffn_kernel(beidx_ref, beact_ref, isf_ref, par_ref, fe_ref, hnext_ref,
                x_ref, w1_ref, b1_ref, w2_ref, b2_ref, prob_ref, y_ref,
                w1b_ref, w2b_ref, wsem):
    nb = pl.program_id(0)
    act = beact_ref[nb] > 0

    def issue(e, p):
        for ch in range(8):
            pltpu.make_async_copy(
                w1_ref.at[e, pl.ds(ch * W1CH, W1CH), :],
                w1b_ref.at[p, pl.ds(ch * W1CH, W1CH), :], wsem).start()
            pltpu.make_async_copy(
                w2_ref.at[e, pl.ds(ch * W2CH, W2CH), :],
                w2b_ref.at[p, pl.ds(ch * W2CH, W2CH), :], wsem).start()

    def drain(e, p):
        for ch in range(8):
            pltpu.make_async_copy(
                w1_ref.at[e, pl.ds(ch * W1CH, W1CH), :],
                w1b_ref.at[p, pl.ds(ch * W1CH, W1CH), :], wsem).wait()
            pltpu.make_async_copy(
                w2_ref.at[e, pl.ds(ch * W2CH, W2CH), :],
                w2b_ref.at[p, pl.ds(ch * W2CH, W2CH), :], wsem).wait()

    @pl.when(jnp.logical_and(act, nb == 0))
    def _():
        issue(beidx_ref[0], par_ref[0])

    @pl.when(jnp.logical_and(act, isf_ref[nb] > 0))
    def _():
        e = beidx_ref[nb]
        p = par_ref[nb]
        drain(e, p)

        @pl.when(hnext_ref[nb] > 0)
        def _():
            issue(fe_ref[nb], 1 - p)

    @pl.when(act)
    def _():
        p = par_ref[nb]
        h = jnp.dot(x_ref[...], w1b_ref[p], preferred_element_type=jnp.float32)
        h = h + b1_ref[0, 0, :]
        h = 0.5 * h * (1.0 + lax.erf(h * 0.7071067811865476))
        y = jnp.dot(h, w2b_ref[p],
                    preferred_element_type=jnp.float32) + b2_ref[0, 0, :]
        y_ref[...] = y * prob_ref[:, 0:1]


def _ffn(beidx, beact, isf, par, fe, hnext, xs, w1f, b1r, w2f, b2r, ps):
    grid_spec = pltpu.PrefetchScalarGridSpec(
        num_scalar_prefetch=6,
        grid=(NB,),
        in_specs=[
            pl.BlockSpec((BLK, D_MODEL), lambda nb, *_: (nb, 0)),
            pl.BlockSpec(memory_space=pl.ANY),
            pl.BlockSpec((1, 1, D_FF), lambda nb, bi, *_: (bi[nb], 0, 0)),
            pl.BlockSpec(memory_space=pl.ANY),
            pl.BlockSpec((1, 1, D_MODEL), lambda nb, bi, *_: (bi[nb], 0, 0)),
            pl.BlockSpec((BLK, 128), lambda nb, *_: (nb, 0)),
        ],
        out_specs=pl.BlockSpec((BLK, D_MODEL), lambda nb, *_: (nb, 0)),
        scratch_shapes=[
            pltpu.VMEM((2, D_MODEL, D_FF), jnp.float32),
            pltpu.VMEM((2, D_FF, D_MODEL), jnp.float32),
            pltpu.SemaphoreType.DMA,
        ],
    )
    return pl.pallas_call(
        _ffn_kernel,
        grid_spec=grid_spec,
        out_shape=jax.ShapeDtypeStruct((NSORT, D_MODEL), jnp.float32),
    )(beidx, beact, isf, par, fe, hnext, xs, w1f, b1r, w2f, b2r, ps)


@functools.lru_cache(maxsize=None)
def _make_combine():
    mesh = plsc.VectorSubcoreMesh(core_axis_name="c", subcore_axis_name="s")

    nch = ROWS_D // CHD

    @functools.partial(
        pl.kernel,
        mesh=mesh,
        out_type=jax.ShapeDtypeStruct((T, D_MODEL), jnp.float32),
        scratch_types=(
            [pltpu.VMEM((CHD,), jnp.int32)] * (2 * NSETS)
            + [pltpu.VMEM((CHD, D_MODEL), jnp.float32)] * (2 * NSETS)
            + [pltpu.SemaphoreType.DMA] * NSETS
            + [pltpu.SemaphoreType.DMA] * NSETS
        ),
    )
    def combine_k(y_hbm, pos_hbm, o_hbm, *scr):
        i0_v = scr[0:NSETS]
        i1_v = scr[NSETS:2 * NSETS]
        g0_v = scr[2 * NSETS:3 * NSETS]
        g1_v = scr[3 * NSETS:4 * NSETS]
        gsem = scr[4 * NSETS:5 * NSETS]
        ssem = scr[5 * NSETS:6 * NSETS]
        wid = lax.axis_index("s") * NC + lax.axis_index("c")
        tok_base = wid * ROWS_D

        def gather(c):
            s = c % NSETS
            tb = tok_base + c * CHD
            pltpu.sync_copy(pos_hbm.at[pl.ds(tb, CHD)], i0_v[s])
            pltpu.sync_copy(pos_hbm.at[pl.ds(T + tb, CHD)], i1_v[s])
            return (
                pltpu.async_copy(y_hbm.at[i0_v[s]], g0_v[s], gsem[s]),
                pltpu.async_copy(y_hbm.at[i1_v[s]], g1_v[s], gsem[s]),
            )

        gathers = {c: gather(c) for c in range(NSETS)}
        stores = {}
        for c in range(nch):
            s = c % NSETS
            tb = tok_base + c * CHD
            for cp in gathers[c]:
                cp.wait()

            @pl.loop(0, CHD)
            def _(r):
                @pl.loop(0, D_MODEL, step=16)
                def _(cc):
                    sl = (pl.ds(r, 1), pl.ds(cc, 16))
                    g0_v[s].at[*sl][...] = (g0_v[s].at[*sl][...]
                                            + g1_v[s].at[*sl][...])

            stores[c] = pltpu.async_copy(g0_v[s], o_hbm.at[pl.ds(tb, CHD)],
                                         ssem[s])
            if c + NSETS < nch:
                stores[c].wait()
                stores.pop(c)
                gathers[c + NSETS] = gather(c + NSETS)
        for c in stores:
            stores[c].wait()

    return combine_k


def kernel(x, W1, b1, W2, b2, Wg, bg):
    B, S, D = x.shape
    x2d = x.reshape(T, D)
    b1r = b1.reshape(N_EXP, 1, D_FF)
    b2r = b2.reshape(N_EXP, 1, D_MODEL)
    bg2 = bg.reshape(1, N_EXP)

    (pos, prob, beidx, beact, isf, par, fe, hnext) = _gating(x2d, Wg, bg2)
    pos1d = pos.reshape(NS)
    xs, ps = _make_scatter()(x2d, pos1d, prob)
    ys = _ffn(beidx.reshape(NB), beact.reshape(NB), isf.reshape(NB),
              par.reshape(NB), fe.reshape(NB), hnext.reshape(NB), xs, W1,
              b1r, W2, b2r, ps)
    out = _make_combine()(ys, pos1d)
    return out.reshape(B, S, D)


# speedup vs baseline: 1.0808x; 1.0424x over previous
import functools

import jax
import jax.numpy as jnp
from jax import lax
from jax.experimental import pallas as pl
from jax.experimental.pallas import tpu as pltpu
from jax.experimental.pallas import tpu_sc as plsc

D_MODEL = 1024
D_FF = 2048
N_EXP = 8
T = 2048
K = 2
NS = 4096
BLK = 320
NB = 20
NSORT = NB * BLK
NEG = -1e30

NC = 2
NSUB = 16
NW = NC * NSUB
ROWS_B = NS // NW
ROWS_D = T // NW
CHB = 32
CHD = 16
NSETS = 3


def _cumsum_rows(a):
    n = a.shape[0]
    s = 1
    while s < n:
        sh = jnp.concatenate(
            [jnp.zeros((s, a.shape[1]), a.dtype), a[:n - s]], axis=0)
        a = a + sh
        s *= 2
    return a


def _cumsum_lanes(a):
    n = a.shape[1]
    s = 1
    while s < n:
        sh = jnp.concatenate(
            [jnp.zeros((a.shape[0], s), a.dtype), a[:, :n - s]], axis=1)
        a = a + sh
        s *= 2
    return a


def _gating_kernel(x_ref, wg_ref, bg_ref, pos_ref, prob_ref, beidx_ref,
                   beact_ref, isf_ref, par_ref, fe_ref, hnext_ref):
    g = jnp.dot(x_ref[...], wg_ref[...],
                preferred_element_type=jnp.float32) + bg_ref[0, :]
    idx8 = lax.broadcasted_iota(jnp.int32, (T, N_EXP), 1)
    m0 = jnp.max(g, axis=1, keepdims=True)
    e0 = jnp.min(jnp.where(g >= m0, idx8, N_EXP), axis=1, keepdims=True)
    gm = jnp.where(idx8 == e0, NEG, g)
    m1 = jnp.max(gm, axis=1, keepdims=True)
    e1 = jnp.min(jnp.where(gm >= m1, idx8, N_EXP), axis=1, keepdims=True)
    p0 = 1.0 / (1.0 + jnp.exp(m1 - m0))
    p1 = 1.0 - p0

    oh0 = (idx8 == e0).astype(jnp.float32)
    oh1 = (idx8 == e1).astype(jnp.float32)
    c01 = _cumsum_rows(jnp.concatenate([oh0, oh1], axis=1))
    c0 = c01[:, :N_EXP]
    c1 = c01[:, N_EXP:]
    cnt0 = c0[T - 1:T, :]
    cnt = cnt0 + c1[T - 1:T, :]
    padded = jnp.ceil(cnt * (1.0 / BLK)) * BLK
    off = _cumsum_lanes(padded) - padded

    r0 = jnp.sum(oh0 * (off + c0), axis=1, keepdims=True) - 1.0
    r1 = jnp.sum(oh1 * (off + cnt0 + c1), axis=1, keepdims=True) - 1.0
    pos_ref[0:T, :] = r0.astype(jnp.int32)
    pos_ref[T:NS, :] = r1.astype(jnp.int32)

    prob_ref[0:T, :] = jnp.broadcast_to(p0, (T, 128))
    prob_ref[T:NS, :] = jnp.broadcast_to(p1, (T, 128))

    ends = off + padded
    total = jnp.sum(padded, axis=1, keepdims=True)
    nbv = lax.broadcasted_iota(jnp.int32, (NB, 1), 0).astype(jnp.float32) * BLK
    be = jnp.sum((nbv >= ends).astype(jnp.float32), axis=1, keepdims=True)
    act = (nbv < total)
    bi = jnp.minimum(be, N_EXP - 1)
    beidx_ref[...] = bi.astype(jnp.int32)
    beact_ref[...] = act.astype(jnp.int32)

    iota_e = lax.broadcasted_iota(jnp.int32, (NB, N_EXP), 1).astype(jnp.float32)
    oh_b = (iota_e == bi).astype(jnp.float32)
    off_b = jnp.sum(oh_b * off, axis=1, keepdims=True)
    nonempty = (padded > 0).astype(jnp.float32)
    rid_of_f = _cumsum_lanes(nonempty) - nonempty
    rid_b = jnp.sum(oh_b * rid_of_f, axis=1, keepdims=True)
    nruns = jnp.sum(nonempty, axis=1, keepdims=True)
    eidx = lax.broadcasted_iota(jnp.int32, (1, N_EXP), 1).astype(jnp.float32)
    fe = jnp.sum((rid_of_f == rid_b + 1.0) * nonempty * eidx, axis=1,
                 keepdims=True)
    isf_ref[...] = ((nbv == off_b).astype(jnp.float32)
                    * act.astype(jnp.float32)).astype(jnp.int32)
    par_ref[...] = (rid_b - 2.0 * jnp.floor(rid_b * 0.5)).astype(jnp.int32)
    fe_ref[...] = fe.astype(jnp.int32)
    hnext_ref[...] = (rid_b + 1.0 < nruns).astype(jnp.int32)


def _gating(x2d, Wg, bg2):
    return pl.pallas_call(
        _gating_kernel,
        grid=(1,),
        in_specs=[
            pl.BlockSpec((T, D_MODEL), lambda i: (0, 0)),
            pl.BlockSpec((D_MODEL, N_EXP), lambda i: (0, 0)),
            pl.BlockSpec((1, N_EXP), lambda i: (0, 0)),
        ],
        out_specs=[
            pl.BlockSpec((NS, 1), lambda i: (0, 0)),
            pl.BlockSpec((NS, 128), lambda i: (0, 0)),
            pl.BlockSpec((NB, 1), lambda i: (0, 0)),
            pl.BlockSpec((NB, 1), lambda i: (0, 0)),
            pl.BlockSpec((NB, 1), lambda i: (0, 0)),
            pl.BlockSpec((NB, 1), lambda i: (0, 0)),
            pl.BlockSpec((NB, 1), lambda i: (0, 0)),
            pl.BlockSpec((NB, 1), lambda i: (0, 0)),
        ],
        out_shape=[
            jax.ShapeDtypeStruct((NS, 1), jnp.int32),
            jax.ShapeDtypeStruct((NS, 128), jnp.float32),
        ] + [jax.ShapeDtypeStruct((NB, 1), jnp.int32)] * 6,
    )(x2d, Wg, bg2)


@functools.lru_cache(maxsize=None)
def _make_scatter():
    mesh = plsc.VectorSubcoreMesh(core_axis_name="c", subcore_axis_name="s")

    nch = ROWS_B // CHB

    @functools.partial(
        pl.kernel,
        mesh=mesh,
        out_type=[
            jax.ShapeDtypeStruct((NSORT, D_MODEL), jnp.float32),
            jax.ShapeDtypeStruct((NSORT, 128), jnp.float32),
        ],
        scratch_types=(
            [pltpu.VMEM((CHB,), jnp.int32)] * NSETS
            + [pltpu.VMEM((CHB, D_MODEL), jnp.float32)] * NSETS
            + [pltpu.VMEM((CHB, 128), jnp.float32)] * NSETS
            + [pltpu.SemaphoreType.DMA] * NSETS
            + [pltpu.SemaphoreType.DMA] * NSETS
        ),
    )
    def scatter_k(x_hbm, pos_hbm, prob_hbm, xs_hbm, ps_hbm, *scr):
        idx_v = scr[0:NSETS]
        row_v = scr[NSETS:2 * NSETS]
        prb_v = scr[2 * NSETS:3 * NSETS]
        lsem = scr[3 * NSETS:4 * NSETS]
        ssem = scr[4 * NSETS:5 * NSETS]
        wid = lax.axis_index("s") * NC + lax.axis_index("c")
        slot_base = wid * ROWS_B
        tok_base = (wid % NSUB) * ROWS_B

        def load(c):
            s = c % NSETS
            sb = slot_base + c * CHB
            tb = tok_base + c * CHB
            return (
                pltpu.async_copy(pos_hbm.at[pl.ds(sb, CHB)], idx_v[s],
                                 lsem[s]),
                pltpu.async_copy(x_hbm.at[pl.ds(tb, CHB)], row_v[s], lsem[s]),
                pltpu.async_copy(prob_hbm.at[pl.ds(sb, CHB)], prb_v[s],
                                 lsem[s]),
            )

        def store(c):
            s = c % NSETS
            return (
                pltpu.async_copy(row_v[s], xs_hbm.at[idx_v[s]], ssem[s]),
                pltpu.async_copy(prb_v[s], ps_hbm.at[idx_v[s]], ssem[s]),
            )

        loads = {c: load(c) for c in range(NSETS)}
        stores = {}
        for c in range(nch):
            for cp in loads[c]:
                cp.wait()
            stores[c] = store(c)
            if c + NSETS < nch:
                for cp in stores[c]:
                    cp.wait()
                stores.pop(c)
                loads[c + NSETS] = load(c + NSETS)
        for c in stores:
            for cp in stores[c]:
                cp.wait()

    return scatter_k


W1CH = D_MODEL // 8
W2CH = D_FF // 8


def _ffn_kernel(beidx_ref, beact_ref, isf_ref, par_ref, fe_ref, hnext_ref,
                x_ref, w1_ref, b1_ref, w2_ref, b2_ref, prob_ref, y_ref,
                w1b_ref, w2b_ref, wsem):
    nb = pl.program_id(0)
    act = beact_ref[nb] > 0

    def issue(e, p):
        for ch in range(8):
            pltpu.make_async_copy(
                w1_ref.at[e, pl.ds(ch * W1CH, W1CH), :],
                w1b_ref.at[p, pl.ds(ch * W1CH, W1CH), :], wsem).start()
            pltpu.make_async_copy(
                w2_ref.at[e, pl.ds(ch * W2CH, W2CH), :],
                w2b_ref.at[p, pl.ds(ch * W2CH, W2CH), :], wsem).start()

    def drain(e, p):
        for ch in range(8):
            pltpu.make_async_copy(
                w1_ref.at[e, pl.ds(ch * W1CH, W1CH), :],
                w1b_ref.at[p, pl.ds(ch * W1CH, W1CH), :], wsem).wait()
            pltpu.make_async_copy(
                w2_ref.at[e, pl.ds(ch * W2CH, W2CH), :],
                w2b_ref.at[p, pl.ds(ch * W2CH, W2CH), :], wsem).wait()

    @pl.when(jnp.logical_and(act, nb == 0))
    def _():
        issue(beidx_ref[0], par_ref[0])

    @pl.when(jnp.logical_and(act, isf_ref[nb] > 0))
    def _():
        e = beidx_ref[nb]
        p = par_ref[nb]
        drain(e, p)

        @pl.when(hnext_ref[nb] > 0)
        def _():
            issue(fe_ref[nb], 1 - p)

    @pl.when(act)
    def _():
        p = par_ref[nb]
        h = jnp.dot(x_ref[...], w1b_ref[p], preferred_element_type=jnp.float32)
        h = h + b1_ref[0, 0, :]
        h = 0.5 * h * (1.0 + lax.erf(h * 0.7071067811865476))
        y = jnp.dot(h, w2b_ref[p],
                    preferred_element_type=jnp.float32) + b2_ref[0, 0, :]
        y_ref[...] = y * prob_ref[:, 0:1]


def _ffn(beidx, beact, isf, par, fe, hnext, xs, w1f, b1r, w2f, b2r, ps):
    grid_spec = pltpu.PrefetchScalarGridSpec(
        num_scalar_prefetch=6,
        grid=(NB,),
        in_specs=[
            pl.BlockSpec((BLK, D_MODEL), lambda nb, *_: (nb, 0)),
            pl.BlockSpec(memory_space=pl.ANY),
            pl.BlockSpec((1, 1, D_FF), lambda nb, bi, *_: (bi[nb], 0, 0)),
            pl.BlockSpec(memory_space=pl.ANY),
            pl.BlockSpec((1, 1, D_MODEL), lambda nb, bi, *_: (bi[nb], 0, 0)),
            pl.BlockSpec((BLK, 128), lambda nb, *_: (nb, 0)),
        ],
        out_specs=pl.BlockSpec((BLK, D_MODEL), lambda nb, *_: (nb, 0)),
        scratch_shapes=[
            pltpu.VMEM((2, D_MODEL, D_FF), jnp.float32),
            pltpu.VMEM((2, D_FF, D_MODEL), jnp.float32),
            pltpu.SemaphoreType.DMA,
        ],
    )
    return pl.pallas_call(
        _ffn_kernel,
        grid_spec=grid_spec,
        out_shape=jax.ShapeDtypeStruct((NSORT, D_MODEL), jnp.float32),
    )(beidx, beact, isf, par, fe, hnext, xs, w1f, b1r, w2f, b2r, ps)


@functools.lru_cache(maxsize=None)
def _make_combine():
    mesh = plsc.VectorSubcoreMesh(core_axis_name="c", subcore_axis_name="s")

    nch = ROWS_D // CHD

    @functools.partial(
        pl.kernel,
        mesh=mesh,
        out_type=jax.ShapeDtypeStruct((T, D_MODEL), jnp.float32),
        scratch_types=(
            [pltpu.VMEM((CHD,), jnp.int32)] * (2 * NSETS)
            + [pltpu.VMEM((CHD, D_MODEL), jnp.float32)] * (2 * NSETS)
            + [pltpu.SemaphoreType.DMA] * NSETS
            + [pltpu.SemaphoreType.DMA] * NSETS
        ),
    )
    def combine_k(y_hbm, pos_hbm, o_hbm, *scr):
        i0_v = scr[0:NSETS]
        i1_v = scr[NSETS:2 * NSETS]
        g0_v = scr[2 * NSETS:3 * NSETS]
        g1_v = scr[3 * NSETS:4 * NSETS]
        gsem = scr[4 * NSETS:5 * NSETS]
        ssem = scr[5 * NSETS:6 * NSETS]
        wid = lax.axis_index("s") * NC + lax.axis_index("c")
        tok_base = wid * ROWS_D

        def gather(c):
            s = c % NSETS
            tb = tok_base + c * CHD
            pltpu.sync_copy(pos_hbm.at[pl.ds(tb, CHD)], i0_v[s])
            pltpu.sync_copy(pos_hbm.at[pl.ds(T + tb, CHD)], i1_v[s])
            return (
                pltpu.async_copy(y_hbm.at[i0_v[s]], g0_v[s], gsem[s]),
                pltpu.async_copy(y_hbm.at[i1_v[s]], g1_v[s], gsem[s]),
            )

        gathers = {c: gather(c) for c in range(NSETS)}
        stores = {}
        for c in range(nch):
            s = c % NSETS
            tb = tok_base + c * CHD
            for cp in gathers[c]:
                cp.wait()

            @pl.loop(0, CHD)
            def _(r):
                @pl.loop(0, D_MODEL, step=16)
                def _(cc):
                    sl = (pl.ds(r, 1), pl.ds(cc, 16))
                    g0_v[s].at[*sl][...] = (g0_v[s].at[*sl][...]
                                            + g1_v[s].at[*sl][...])

            stores[c] = pltpu.async_copy(g0_v[s], o_hbm.at[pl.ds(tb, CHD)],
                                         ssem[s])
            if c + NSETS < nch:
                stores[c].wait()
                stores.pop(c)
                gathers[c + NSETS] = gather(c + NSETS)
        for c in stores:
            stores[c].wait()

    return combine_k


def kernel(x, W1, b1, W2, b2, Wg, bg):
    B, S, D = x.shape
    x2d = x.reshape(T, D)
    b1r = b1.reshape(N_EXP, 1, D_FF)
    b2r = b2.reshape(N_EXP, 1, D_MODEL)
    bg2 = bg.reshape(1, N_EXP)

    (pos, prob, beidx, beact, isf, par, fe, hnext) = _gating(x2d, Wg, bg2)
    pos1d = pos.reshape(NS)
    xs, ps = _make_scatter()(x2d, pos1d, prob)
    ys = _ffn(beidx.reshape(NB), beact.reshape(NB), isf.reshape(NB),
              par.reshape(NB), fe.reshape(NB), hnext.reshape(NB), xs, W1,
              b1r, W2, b2r, ps)
    out = _make_combine()(ys, pos1d)
    return out.reshape(B, S, D)
